# initial kernel scaffold (unmeasured)
import jax
import jax.numpy as jnp
from jax import lax
from jax.experimental import pallas as pl
from jax.experimental.pallas import tpu as pltpu

N_DEV = 4
N_PASS = 2
CM = 512
CT = 2048
N_HOPS = 6 * N_PASS


def _gelu(v):
    c = 0.7978845608028654
    return 0.5 * v * (1.0 + jnp.tanh(c * (v + 0.044715 * v * v * v)))


def kernel(x, w_mat):
    m, k = x.shape
    _, n = w_mat.shape
    n_tiles = n // CT

    def body(x_ref, w_ref, out_ref, comm_ref, send_sems, recv_sems,
             credit_sem, copy_sem):
        my = lax.axis_index("i")
        left = lax.rem(my + (N_DEV - 1), N_DEV)
        right = lax.rem(my + 1, N_DEV)

        barrier_sem = pltpu.get_barrier_semaphore()
        for nbr in (left, right):
            pl.semaphore_signal(
                barrier_sem, inc=1,
                device_id=(nbr,), device_id_type=pl.DeviceIdType.MESH,
            )
        pl.semaphore_wait(barrier_sem, 2)

        def partial_tile(q, j):
            return jnp.dot(
                x_ref[pl.ds(q * CM, CM), :],
                w_ref[:, j * CT:(j + 1) * CT],
                preferred_element_type=jnp.float32,
            )

        def init_slot(q, s):
            for j in range(n_tiles):
                comm_ref[s, :, j * CT:(j + 1) * CT] = (
                    partial_tile(q, j).astype(jnp.bfloat16))

        def accumulate(q, r, final):
            for j in range(n_tiles):
                acc = (comm_ref[r, :, j * CT:(j + 1) * CT]
                       .astype(jnp.float32) + partial_tile(q, j))
                if final:
                    acc = _gelu(acc)
                comm_ref[r, :, j * CT:(j + 1) * CT] = acc.astype(jnp.bfloat16)

        def dma_out(q, r):
            cp = pltpu.make_async_copy(
                comm_ref.at[r], out_ref.at[pl.ds(q * CM, CM), :], copy_sem)
            cp.start()
            cp.wait()

        for g in range(N_HOPS):
            p, h = divmod(g, 6)
            s, r = g % 2, (g + 1) % 2

            if h == 0:
                q0 = p * N_DEV + lax.rem(my + 3, N_DEV)
                init_slot(q0, s)

            if g >= 1:
                pl.semaphore_wait(credit_sem, 1)

            rdma = pltpu.make_async_remote_copy(
                src_ref=comm_ref.at[s],
                dst_ref=comm_ref.at[r],
                send_sem=send_sems.at[s],
                recv_sem=recv_sems.at[r],
                device_id=(right,),
                device_id_type=pl.DeviceIdType.MESH,
            )
            rdma.start()
            rdma.wait()

            if g <= N_HOPS - 2:
                pl.semaphore_signal(
                    credit_sem, inc=1,
                    device_id=(left,), device_id_type=pl.DeviceIdType.MESH,
                )

            if h < 3:
                q = p * N_DEV + lax.rem(my + (2 - h) + N_DEV, N_DEV)
                accumulate(q, r, final=(h == 2))
                if h == 2:
                    dma_out(q, r)
            else:
                t = h - 3
                q = p * N_DEV + lax.rem(my + (3 - t), N_DEV)
                dma_out(q, r)

    return pl.pallas_call(
        body,
        out_shape=jax.ShapeDtypeStruct((m, n), jnp.bfloat16),
        in_specs=[
            pl.BlockSpec(memory_space=pltpu.VMEM),
            pl.BlockSpec(memory_space=pltpu.VMEM),
        ],
        out_specs=pl.BlockSpec(memory_space=pltpu.ANY),
        scratch_shapes=[
            pltpu.VMEM((2, CM, n), jnp.bfloat16),
            pltpu.SemaphoreType.DMA((2,)),
            pltpu.SemaphoreType.DMA((2,)),
            pltpu.SemaphoreType.REGULAR,
            pltpu.SemaphoreType.DMA,
        ],
        compiler_params=pltpu.CompilerParams(collective_id=0),
    )(x, w_mat)


# baseline (device time: 1287827 ns/iter reference)
import jax
import jax.numpy as jnp
from jax import lax
from jax.experimental import pallas as pl
from jax.experimental.pallas import tpu as pltpu

N_DEV = 4
N_PASS = 2
CM = 512
CT = 1024
N_HOPS = 6 * N_PASS
_USE_CREDITS = False


def _gelu(v):
    c = 0.7978845608028654
    return 0.5 * v * (1.0 + jnp.tanh(c * (v + 0.044715 * v * v * v)))


def kernel(x, w_mat):
    x = x.astype(jnp.bfloat16)
    w_mat = w_mat.astype(jnp.bfloat16)
    m, k = x.shape
    _, n = w_mat.shape
    n_tiles = n // CT

    def body(x_ref, w_ref, out_ref, comm_ref, send_sems, recv_sems,
             credit_sem, copy_sem):
        my = lax.axis_index("i")
        left = lax.rem(my + (N_DEV - 1), N_DEV)
        right = lax.rem(my + 1, N_DEV)

        barrier_sem = pltpu.get_barrier_semaphore()
        for nbr in (left, right):
            pl.semaphore_signal(
                barrier_sem, inc=1,
                device_id=(nbr,), device_id_type=pl.DeviceIdType.MESH,
            )
        pl.semaphore_wait(barrier_sem, 2)

        def partial_tile(q, j):
            return jnp.dot(
                x_ref[pl.ds(q * CM, CM), :],
                w_ref[:, j * CT:(j + 1) * CT],
                preferred_element_type=jnp.float32,
            )

        def init_slot(q, s):
            for j in range(n_tiles):
                comm_ref[s, :, j * CT:(j + 1) * CT] = (
                    partial_tile(q, j).astype(jnp.bfloat16))

        def accumulate(q, r, final):
            for j in range(n_tiles):
                acc = (comm_ref[r, :, j * CT:(j + 1) * CT]
                       .astype(jnp.float32) + partial_tile(q, j))
                if final:
                    acc = _gelu(acc)
                comm_ref[r, :, j * CT:(j + 1) * CT] = acc.astype(jnp.bfloat16)

        def dma_out(q, r):
            cp = pltpu.make_async_copy(
                comm_ref.at[r], out_ref.at[pl.ds(q * CM, CM), :], copy_sem)
            cp.start()
            cp.wait()

        for g in range(N_HOPS):
            p, h = divmod(g, 6)
            s, r = g % 2, (g + 1) % 2

            if h == 0:
                q0 = p * N_DEV + lax.rem(my + 3, N_DEV)
                init_slot(q0, s)

            if g >= 1 and _USE_CREDITS:
                pl.semaphore_wait(credit_sem, 1)

            rdma = pltpu.make_async_remote_copy(
                src_ref=comm_ref.at[s],
                dst_ref=comm_ref.at[r],
                send_sem=send_sems.at[s],
                recv_sem=recv_sems.at[r],
                device_id=(right,),
                device_id_type=pl.DeviceIdType.MESH,
            )
            rdma.start()
            rdma.wait()

            if g <= N_HOPS - 2 and _USE_CREDITS:
                pl.semaphore_signal(
                    credit_sem, inc=1,
                    device_id=(left,), device_id_type=pl.DeviceIdType.MESH,
                )

            if h < 3:
                q = p * N_DEV + lax.rem(my + (2 - h) + N_DEV, N_DEV)
                accumulate(q, r, final=(h == 2))
                if h == 2:
                    dma_out(q, r)
            else:
                t = h - 3
                q = p * N_DEV + lax.rem(my + (3 - t), N_DEV)
                dma_out(q, r)

    return pl.pallas_call(
        body,
        out_shape=jax.ShapeDtypeStruct((m, n), jnp.bfloat16),
        in_specs=[
            pl.BlockSpec(memory_space=pltpu.VMEM),
            pl.BlockSpec(memory_space=pltpu.VMEM),
        ],
        out_specs=pl.BlockSpec(memory_space=pl.ANY),
        scratch_shapes=[
            pltpu.VMEM((2, CM, n), jnp.bfloat16),
            pltpu.SemaphoreType.DMA((2,)),
            pltpu.SemaphoreType.DMA((2,)),
            pltpu.SemaphoreType.REGULAR,
            pltpu.SemaphoreType.DMA,
        ],
        compiler_params=pltpu.CompilerParams(
            collective_id=0, vmem_limit_bytes=38 * 1024 * 1024),
    )(x, w_mat)


# device time: 685778 ns/iter; 1.8779x vs baseline; 1.8779x over previous
import jax
import jax.numpy as jnp
from jax import lax
from jax.experimental import pallas as pl
from jax.experimental.pallas import tpu as pltpu

N_DEV = 4
N_PASS = 2
CM = 256
CT = 2048
N_HOPS = 6 * N_PASS


def _gelu(v):
    c = 0.7978845608028654
    return 0.5 * v * (1.0 + jnp.tanh(c * (v + 0.044715 * v * v * v)))


def kernel(x, w_mat):
    x = x.astype(jnp.bfloat16)
    w_mat = w_mat.astype(jnp.bfloat16)
    m, k = x.shape
    _, n = w_mat.shape
    n_tiles = n // CT
    half_chunks = m // (2 * CM)

    def body(x_ref, w_ref, out_ref, comm_r, comm_l, p_buf,
             send_sems_r, recv_sems_r, send_sems_l, recv_sems_l,
             copy_sem_r, copy_sem_l):
        my = lax.axis_index("i")
        left = lax.rem(my + (N_DEV - 1), N_DEV)
        right = lax.rem(my + 1, N_DEV)

        barrier_sem = pltpu.get_barrier_semaphore()
        for nbr in (left, right):
            pl.semaphore_signal(
                barrier_sem, inc=1,
                device_id=(nbr,), device_id_type=pl.DeviceIdType.MESH,
            )
        pl.semaphore_wait(barrier_sem, 2)

        def row_base(d, p, q_ring):
            return (d * N_PASS * N_DEV + p * N_DEV + q_ring) * CM

        def partial_to_pbuf(d, rb):
            for j in range(n_tiles):
                p_buf[d, :, j * CT:(j + 1) * CT] = jnp.dot(
                    x_ref[pl.ds(rb, CM), :],
                    w_ref[:, j * CT:(j + 1) * CT],
                    preferred_element_type=jnp.float32,
                ).astype(jnp.bfloat16)

        def init_slot(comm, d, rb, s):
            for j in range(n_tiles):
                comm[s, :, j * CT:(j + 1) * CT] = jnp.dot(
                    x_ref[pl.ds(rb, CM), :],
                    w_ref[:, j * CT:(j + 1) * CT],
                    preferred_element_type=jnp.float32,
                ).astype(jnp.bfloat16)

        def accumulate(comm, d, r, final):
            for j in range(n_tiles):
                acc = (comm[r, :, j * CT:(j + 1) * CT].astype(jnp.float32)
                       + p_buf[d, :, j * CT:(j + 1) * CT].astype(jnp.float32))
                if final:
                    acc = _gelu(acc)
                comm[r, :, j * CT:(j + 1) * CT] = acc.astype(jnp.bfloat16)

        pending = [None, None]

        def dma_out(comm, d, rb, r, sem):
            if pending[d] is not None:
                pending[d].wait()
            cp = pltpu.make_async_copy(
                comm.at[r], out_ref.at[pl.ds(rb, CM), :], sem)
            cp.start()
            pending[d] = cp

        for g in range(N_HOPS):
            p, h = divmod(g, 6)
            s, r = g % 2, (g + 1) % 2

            if h == 0:
                for d in (0, 1):
                    if pending[d] is not None:
                        pending[d].wait()
                        pending[d] = None
                init_slot(comm_r, 0, row_base(0, p, lax.rem(my + 3, N_DEV)), s)
                init_slot(comm_l, 1, row_base(1, p, lax.rem(my + 1, N_DEV)), s)

            rdma_r = pltpu.make_async_remote_copy(
                src_ref=comm_r.at[s], dst_ref=comm_r.at[r],
                send_sem=send_sems_r.at[s], recv_sem=recv_sems_r.at[r],
                device_id=(right,), device_id_type=pl.DeviceIdType.MESH,
            )
            rdma_l = pltpu.make_async_remote_copy(
                src_ref=comm_l.at[s], dst_ref=comm_l.at[r],
                send_sem=send_sems_l.at[s], recv_sem=recv_sems_l.at[r],
                device_id=(left,), device_id_type=pl.DeviceIdType.MESH,
            )
            rdma_r.start()
            rdma_l.start()

            if h < 3:
                q_r = lax.rem(my + (2 - h) + N_DEV, N_DEV)
                q_l = lax.rem(my + 2 + h, N_DEV)
                rb_r = row_base(0, p, q_r)
                rb_l = row_base(1, p, q_l)
                partial_to_pbuf(0, rb_r)
                partial_to_pbuf(1, rb_l)
                rdma_r.wait()
                accumulate(comm_r, 0, r, final=(h == 2))
                if h == 2:
                    dma_out(comm_r, 0, rb_r, r, copy_sem_r)
                rdma_l.wait()
                accumulate(comm_l, 1, r, final=(h == 2))
                if h == 2:
                    dma_out(comm_l, 1, rb_l, r, copy_sem_l)
            else:
                t = h - 3
                rb_r = row_base(0, p, lax.rem(my + 3 - t, N_DEV))
                rb_l = row_base(1, p, lax.rem(my + 1 + t, N_DEV))
                rdma_r.wait()
                dma_out(comm_r, 0, rb_r, r, copy_sem_r)
                rdma_l.wait()
                dma_out(comm_l, 1, rb_l, r, copy_sem_l)

        for d in (0, 1):
            if pending[d] is not None:
                pending[d].wait()

    return pl.pallas_call(
        body,
        out_shape=jax.ShapeDtypeStruct((m, n), jnp.bfloat16),
        in_specs=[
            pl.BlockSpec(memory_space=pltpu.VMEM),
            pl.BlockSpec(memory_space=pltpu.VMEM),
        ],
        out_specs=pl.BlockSpec(memory_space=pl.ANY),
        scratch_shapes=[
            pltpu.VMEM((2, CM, n), jnp.bfloat16),
            pltpu.VMEM((2, CM, n), jnp.bfloat16),
            pltpu.VMEM((2, CM, n), jnp.bfloat16),
            pltpu.SemaphoreType.DMA((2,)),
            pltpu.SemaphoreType.DMA((2,)),
            pltpu.SemaphoreType.DMA((2,)),
            pltpu.SemaphoreType.DMA((2,)),
            pltpu.SemaphoreType.DMA,
            pltpu.SemaphoreType.DMA,
        ],
        compiler_params=pltpu.CompilerParams(
            collective_id=0, vmem_limit_bytes=40 * 1024 * 1024),
    )(x, w_mat)


# device time: 660992 ns/iter; 1.9483x vs baseline; 1.0375x over previous
import jax
import jax.numpy as jnp
from jax import lax
from jax.experimental import pallas as pl
from jax.experimental.pallas import tpu as pltpu

N_DEV = 4
N_PASS = 2
CM = 256
SUB = 2
SR = CM // SUB
CT_DOT = 4096
CT_GELU = 4096


def _gelu(v):
    c = 0.7978845608028654
    return 0.5 * v * (1.0 + jnp.tanh(c * (v + 0.044715 * v * v * v)))


def kernel(x, w_mat):
    x = x.astype(jnp.bfloat16)
    w_mat = w_mat.astype(jnp.bfloat16)
    m, k = x.shape
    _, n = w_mat.shape

    def body(x_ref, w_ref, out_ref, comm_r, comm_l, p_buf,
             send_sems_r, recv_sems_r, send_sems_l, recv_sems_l,
             copy_sems_r, copy_sems_l):
        my = lax.axis_index("i")
        left = lax.rem(my + (N_DEV - 1), N_DEV)
        right = lax.rem(my + 1, N_DEV)

        barrier_sem = pltpu.get_barrier_semaphore()
        for nbr in (left, right):
            pl.semaphore_signal(
                barrier_sem, inc=1,
                device_id=(nbr,), device_id_type=pl.DeviceIdType.MESH,
            )
        pl.semaphore_wait(barrier_sem, 2)

        rings = (
            (comm_r, send_sems_r, recv_sems_r, right, copy_sems_r),
            (comm_l, send_sems_l, recv_sems_l, left, copy_sems_l),
        )

        def row_base(d, p, q_ring):
            return (d * N_PASS * N_DEV + p * N_DEV + q_ring) * CM

        def sub_cell(d, h, u):
            comm, ssems, rsems, dev, _ = rings[d]
            s, r = h % 2, (h + 1) % 2
            return pltpu.make_async_remote_copy(
                src_ref=comm.at[s, u], dst_ref=comm.at[r, u],
                send_sem=ssems.at[s, u], recv_sem=rsems.at[r, u],
                device_id=(dev,), device_id_type=pl.DeviceIdType.MESH,
            )

        def full_cell(d, h):
            comm, ssems, rsems, dev, _ = rings[d]
            s, r = h % 2, (h + 1) % 2
            return pltpu.make_async_remote_copy(
                src_ref=comm.at[s], dst_ref=comm.at[r],
                send_sem=ssems.at[s, 0], recv_sem=rsems.at[r, 0],
                device_id=(dev,), device_id_type=pl.DeviceIdType.MESH,
            )

        def dot_rows(rb, nrows, out_view):
            for j in range(n // CT_DOT):
                out_view[:, j * CT_DOT:(j + 1) * CT_DOT] = jnp.dot(
                    x_ref[pl.ds(rb, nrows), :],
                    w_ref[:, j * CT_DOT:(j + 1) * CT_DOT],
                    preferred_element_type=jnp.float32,
                ).astype(jnp.bfloat16)

        def accumulate(d, r, u, final):
            comm = rings[d][0]
            ct = CT_GELU if final else n
            for j in range(n // ct):
                js = slice(j * ct, (j + 1) * ct)
                acc = (comm[r, u, :, js].astype(jnp.float32)
                       + p_buf[d, u * SR:(u + 1) * SR, js]
                       .astype(jnp.float32))
                if final:
                    acc = _gelu(acc)
                comm[r, u, :, js] = acc.astype(jnp.bfloat16)

        def out_copy(d, rb, r, u):
            comm, _, _, _, csems = rings[d]
            return pltpu.make_async_copy(
                comm.at[r, u], out_ref.at[pl.ds(rb + u * SR, SR), :],
                csems.at[u])

        def dma_out(d, rb, r, u, first):
            if not first:
                out_copy(d, 0, r, u).wait()
            out_copy(d, rb, r, u).start()

        for d in (0, 1):
            for u in (0, 1):
                out_copy(d, row_base(d, 0, 0), 0, u).start()

        def one_pass(p, _):
            for h in range(6):
                s, r = h % 2, (h + 1) % 2

                if h == 0:
                    for d in (0, 1):
                        for u in (0, 1):
                            out_copy(d, 0, 0, u).wait()
                    for d in (0, 1):
                        q0 = lax.rem(my + (3 if d == 0 else 1), N_DEV)
                        rb = row_base(d, p, q0)
                        comm = rings[d][0]
                        for u in range(SUB):
                            dot_rows(rb + u * SR, SR, comm.at[s, u])
                    for u in (0, 1):
                        for d in (0, 1):
                            sub_cell(d, h, u).start()

                if h < 3:
                    q_r = lax.rem(my + (2 - h) + N_DEV, N_DEV)
                    q_l = lax.rem(my + 2 + h, N_DEV)
                    rbs = (row_base(0, p, q_r), row_base(1, p, q_l))
                    for d in (0, 1):
                        dot_rows(rbs[d], CM, p_buf.at[d])
                    for u in (0, 1):
                        for d in (0, 1):
                            sub_cell(d, h, u).wait()
                            accumulate(d, r, u, final=(h == 2))
                            if h < 2:
                                sub_cell(d, h + 1, u).start()
                            elif u == 1:
                                full_cell(d, h + 1).start()
                            if h == 2:
                                dma_out(d, rbs[d], r, u, first=True)
                else:
                    t = h - 3
                    rbs = (row_base(0, p, lax.rem(my + 3 - t, N_DEV)),
                           row_base(1, p, lax.rem(my + 1 + t, N_DEV)))
                    for d in (0, 1):
                        full_cell(d, h).wait()
                        if h < 5:
                            full_cell(d, h + 1).start()
                        for u in (0, 1):
                            dma_out(d, rbs[d], r, u, first=False)
            return _

        lax.fori_loop(0, N_PASS, one_pass, None)

        for d in (0, 1):
            for u in (0, 1):
                out_copy(d, 0, 1, u).wait()

    return pl.pallas_call(
        body,
        out_shape=jax.ShapeDtypeStruct((m, n), jnp.bfloat16),
        in_specs=[
            pl.BlockSpec(memory_space=pltpu.VMEM),
            pl.BlockSpec(memory_space=pltpu.VMEM),
        ],
        out_specs=pl.BlockSpec(memory_space=pl.ANY),
        scratch_shapes=[
            pltpu.VMEM((2, SUB, SR, n), jnp.bfloat16),
            pltpu.VMEM((2, SUB, SR, n), jnp.bfloat16),
            pltpu.VMEM((2, CM, n), jnp.bfloat16),
            pltpu.SemaphoreType.DMA((2, SUB)),
            pltpu.SemaphoreType.DMA((2, SUB)),
            pltpu.SemaphoreType.DMA((2, SUB)),
            pltpu.SemaphoreType.DMA((2, SUB)),
            pltpu.SemaphoreType.DMA((SUB,)),
            pltpu.SemaphoreType.DMA((SUB,)),
        ],
        compiler_params=pltpu.CompilerParams(
            collective_id=0, vmem_limit_bytes=40 * 1024 * 1024),
    )(x, w_mat)


# device time: 653224 ns/iter; 1.9715x vs baseline; 1.0119x over previous
import jax
import jax.numpy as jnp
from jax import lax
from jax.experimental import pallas as pl
from jax.experimental.pallas import tpu as pltpu

N_DEV = 4
N_PASS = 2
CM = 256
SUB = 2
SR = CM // SUB
CT_DOT = 4096
CT_GELU = 4096


def _gelu(v):
    c = 0.7978845608028654
    return 0.5 * v * (1.0 + jnp.tanh(c * (v + 0.044715 * v * v * v)))


def kernel(x, w_mat):
    x = x.astype(jnp.bfloat16)
    w_mat = w_mat.astype(jnp.bfloat16)
    m, k = x.shape
    _, n = w_mat.shape

    def body(x_ref, w_ref, out_ref, comm_r, comm_l, p_buf,
             send_sems_r, recv_sems_r, send_sems_l, recv_sems_l,
             copy_sems_r, copy_sems_l):
        my = lax.axis_index("i")
        left = lax.rem(my + (N_DEV - 1), N_DEV)
        right = lax.rem(my + 1, N_DEV)

        barrier_sem = pltpu.get_barrier_semaphore()
        for nbr in (left, right):
            pl.semaphore_signal(
                barrier_sem, inc=1,
                device_id=(nbr,), device_id_type=pl.DeviceIdType.MESH,
            )
        pl.semaphore_wait(barrier_sem, 2)

        rings = (
            (comm_r, send_sems_r, recv_sems_r, right, copy_sems_r),
            (comm_l, send_sems_l, recv_sems_l, left, copy_sems_l),
        )

        def row_base(d, p, q_ring):
            return (d * N_PASS * N_DEV + p * N_DEV + q_ring) * CM

        def sub_cell(d, h, u):
            comm, ssems, rsems, dev, _ = rings[d]
            s, r = h % 2, (h + 1) % 2
            return pltpu.make_async_remote_copy(
                src_ref=comm.at[s, u], dst_ref=comm.at[r, u],
                send_sem=ssems.at[s, u], recv_sem=rsems.at[r, u],
                device_id=(dev,), device_id_type=pl.DeviceIdType.MESH,
            )

        def full_cell(d, h):
            comm, ssems, rsems, dev, _ = rings[d]
            s, r = h % 2, (h + 1) % 2
            return pltpu.make_async_remote_copy(
                src_ref=comm.at[s], dst_ref=comm.at[r],
                send_sem=ssems.at[s, 0], recv_sem=rsems.at[r, 0],
                device_id=(dev,), device_id_type=pl.DeviceIdType.MESH,
            )

        def dot_rows(rb, nrows, out_view):
            for j in range(n // CT_DOT):
                out_view[:, j * CT_DOT:(j + 1) * CT_DOT] = jnp.dot(
                    x_ref[pl.ds(rb, nrows), :],
                    w_ref[:, j * CT_DOT:(j + 1) * CT_DOT],
                    preferred_element_type=jnp.float32,
                ).astype(jnp.bfloat16)

        def accumulate(d, r, u, final):
            comm = rings[d][0]
            ct = CT_GELU if final else n
            for j in range(n // ct):
                js = slice(j * ct, (j + 1) * ct)
                acc = (comm[r, u, :, js].astype(jnp.float32)
                       + p_buf[d, u * SR:(u + 1) * SR, js]
                       .astype(jnp.float32))
                if final:
                    acc = _gelu(acc)
                comm[r, u, :, js] = acc.astype(jnp.bfloat16)

        def out_copy(d, rb, r, u):
            comm, _, _, _, csems = rings[d]
            return pltpu.make_async_copy(
                comm.at[r, u], out_ref.at[pl.ds(rb + u * SR, SR), :],
                csems.at[u])

        def dma_out(d, rb, r, u, first):
            if not first:
                out_copy(d, 0, r, u).wait()
            out_copy(d, rb, r, u).start()

        for d in (0, 1):
            for u in (0, 1):
                out_copy(d, row_base(d, 0, 0), 0, u).start()

        def one_pass(p, _):
            for h in range(6):
                s, r = h % 2, (h + 1) % 2

                if h == 0:
                    for d in (0, 1):
                        for u in (0, 1):
                            out_copy(d, 0, 0, u).wait()
                    for d in (0, 1):
                        q0 = lax.rem(my + (3 if d == 0 else 1), N_DEV)
                        rb = row_base(d, p, q0)
                        comm = rings[d][0]
                        for u in range(SUB):
                            dot_rows(rb + u * SR, SR, comm.at[s, u])
                    for u in (0, 1):
                        for d in (0, 1):
                            sub_cell(d, h, u).start()

                if h < 3:
                    q_r = lax.rem(my + (2 - h) + N_DEV, N_DEV)
                    q_l = lax.rem(my + 2 + h, N_DEV)
                    rbs = (row_base(0, p, q_r), row_base(1, p, q_l))
                    for d in (0, 1):
                        dot_rows(rbs[d], CM, p_buf.at[d])
                    for u in (0, 1):
                        for d in (0, 1):
                            sub_cell(d, h, u).wait()
                            accumulate(d, r, u, final=(h == 2))
                            if h < 2:
                                sub_cell(d, h + 1, u).start()
                            else:
                                sub_cell(d, h + 1, u).start()
                            if h == 2:
                                dma_out(d, rbs[d], r, u, first=True)
                else:
                    t = h - 3
                    rbs = (row_base(0, p, lax.rem(my + 3 - t, N_DEV)),
                           row_base(1, p, lax.rem(my + 1 + t, N_DEV)))
                    for d in (0, 1):
                        if h == 3:
                            sub_cell(d, h, 0).wait()
                            sub_cell(d, h, 1).wait()
                        else:
                            full_cell(d, h).wait()
                        if h < 5:
                            full_cell(d, h + 1).start()
                        for u in (0, 1):
                            dma_out(d, rbs[d], r, u, first=False)
            return _

        lax.fori_loop(0, N_PASS, one_pass, None)

        for d in (0, 1):
            for u in (0, 1):
                out_copy(d, 0, 1, u).wait()

    return pl.pallas_call(
        body,
        out_shape=jax.ShapeDtypeStruct((m, n), jnp.bfloat16),
        in_specs=[
            pl.BlockSpec(memory_space=pltpu.VMEM),
            pl.BlockSpec(memory_space=pltpu.VMEM),
        ],
        out_specs=pl.BlockSpec(memory_space=pl.ANY),
        scratch_shapes=[
            pltpu.VMEM((2, SUB, SR, n), jnp.bfloat16),
            pltpu.VMEM((2, SUB, SR, n), jnp.bfloat16),
            pltpu.VMEM((2, CM, n), jnp.bfloat16),
            pltpu.SemaphoreType.DMA((2, SUB)),
            pltpu.SemaphoreType.DMA((2, SUB)),
            pltpu.SemaphoreType.DMA((2, SUB)),
            pltpu.SemaphoreType.DMA((2, SUB)),
            pltpu.SemaphoreType.DMA((SUB,)),
            pltpu.SemaphoreType.DMA((SUB,)),
        ],
        compiler_params=pltpu.CompilerParams(
            collective_id=0, vmem_limit_bytes=40 * 1024 * 1024),
    )(x, w_mat)
